# Initial kernel scaffold; baseline (speedup 1.0000x reference)
import jax
import jax.numpy as jnp
from jax.experimental import pallas as pl


def kernel(inputs, sequence_lengths, span_targets, embed_table, ff_W, ff_b, net_W, net_b, out_W, out_b):
    raise NotImplementedError("write your pallas kernel here")


# speedup vs baseline: 11.5371x; 11.5371x over previous
import jax
import jax.numpy as jnp
from jax import lax
from jax.experimental import pallas as pl
from jax.experimental.pallas import tpu as pltpu


def _stage1_kernel(x_ref, w2_ref, emb_ref, wse_ref, ffb_ref, g_ref, wmb_ref):
    g_ref[0] = jnp.dot(x_ref[0], w2_ref[...], preferred_element_type=jnp.float32)
    wmb_ref[...] = (
        jnp.dot(emb_ref[...], wse_ref[...], preferred_element_type=jnp.float32)
        + ffb_ref[...]
    )


def _make_stage2(TL, W, NL):
    def _stage2_kernel(
        seq_ref, bg_ref, eg_ref, wmb_ref, netw_ref, netb_ref, outw_ref,
        outb_ref, tgt_ref, out_ref, loss_ref,
    ):
        b = pl.program_id(0)
        t = pl.program_id(1)

        @pl.when(jnp.logical_and(b == 0, t == 0))
        def _init():
            loss_ref[0, 0] = 0.0

        bg = bg_ref[0]
        netw = netw_ref[...]
        netb = netb_ref[...]
        outw = outw_ref[...]
        outb = outb_ref[...]
        seqlen = seq_ref[0, 0, 0]
        l0 = t * TL
        row = l0 + lax.broadcasted_iota(jnp.int32, (TL, 1), 0)

        acc = jnp.zeros((), jnp.float32)
        ega = eg_ref[0, pl.ds(l0, TL), :]
        egb = eg_ref[0, pl.ds(l0 + TL, 16), :]
        ext = jnp.concatenate([ega, egb], axis=0)
        for w in range(W):
            eg = lax.slice_in_dim(ext, w, w + TL, axis=0)
            h = jnp.maximum(bg + eg + wmb_ref[w : w + 1, :], 0.0)
            h = jnp.maximum(
                jnp.dot(h, netw, preferred_element_type=jnp.float32) + netb, 0.0
            )
            lg = jnp.dot(h, outw, preferred_element_type=jnp.float32) + outb
            out_ref[0, w] = lg
            z = tgt_ref[0, w]
            m = (row + w < seqlen).astype(jnp.float32)
            bce = (
                jnp.maximum(lg, 0.0)
                - lg * z
                + jnp.log1p(jnp.exp(-jnp.abs(lg)))
            )
            acc = acc + jnp.sum(bce * m)
        loss_ref[0, 0] += acc

    return _stage2_kernel


def kernel(inputs, sequence_lengths, span_targets, embed_table, ff_W, ff_b,
           net_W, net_b, out_W, out_b):
    B, L, D = inputs.shape
    W, SE = embed_table.shape
    FF = ff_W.shape[1]
    NET = net_W.shape[1]
    NL = out_W.shape[1]
    TL = 128 if L % 128 == 0 else L
    NT = L // TL

    w2 = jnp.concatenate([ff_W[:D], ff_W[D : 2 * D]], axis=1)
    wse = ff_W[2 * D :]
    ffb2 = ff_b.reshape(1, FF)

    g, wmb = pl.pallas_call(
        _stage1_kernel,
        grid=(B,),
        in_specs=[
            pl.BlockSpec((1, L, D), lambda b: (b, 0, 0)),
            pl.BlockSpec((D, 2 * FF), lambda b: (0, 0)),
            pl.BlockSpec((W, SE), lambda b: (0, 0)),
            pl.BlockSpec((SE, FF), lambda b: (0, 0)),
            pl.BlockSpec((1, FF), lambda b: (0, 0)),
        ],
        out_specs=[
            pl.BlockSpec((1, L, 2 * FF), lambda b: (b, 0, 0)),
            pl.BlockSpec((W, FF), lambda b: (0, 0)),
        ],
        out_shape=[
            jax.ShapeDtypeStruct((B, L, 2 * FF), jnp.float32),
            jax.ShapeDtypeStruct((W, FF), jnp.float32),
        ],
    )(inputs, w2, embed_table, wse, ffb2)

    bg = g[..., :FF]
    eg = g[..., FF:]
    eg_pad = jnp.concatenate(
        [eg, jnp.broadcast_to(eg[:, L - 1 :, :], (B, 16, FF))], axis=1
    )
    LP = L + 16

    seq2 = sequence_lengths.reshape(B, 1, 1).astype(jnp.int32)
    tgt_t = jnp.transpose(span_targets, (0, 2, 1, 3))
    netb2 = net_b.reshape(1, NET)
    outb2 = out_b.reshape(1, NL)

    logits_t, loss = pl.pallas_call(
        _make_stage2(TL, W, NL),
        grid=(B, NT),
        in_specs=[
            pl.BlockSpec((1, 1, 1), lambda b, t: (b, 0, 0), memory_space=pltpu.SMEM),
            pl.BlockSpec((1, TL, FF), lambda b, t: (b, t, 0)),
            pl.BlockSpec((1, LP, FF), lambda b, t: (b, 0, 0)),
            pl.BlockSpec((W, FF), lambda b, t: (0, 0)),
            pl.BlockSpec((FF, NET), lambda b, t: (0, 0)),
            pl.BlockSpec((1, NET), lambda b, t: (0, 0)),
            pl.BlockSpec((NET, NL), lambda b, t: (0, 0)),
            pl.BlockSpec((1, NL), lambda b, t: (0, 0)),
            pl.BlockSpec((1, W, TL, NL), lambda b, t: (b, 0, t, 0)),
        ],
        out_specs=[
            pl.BlockSpec((1, W, TL, NL), lambda b, t: (b, 0, t, 0)),
            pl.BlockSpec((1, 1), lambda b, t: (0, 0), memory_space=pltpu.SMEM),
        ],
        out_shape=[
            jax.ShapeDtypeStruct((B, W, L, NL), jnp.float32),
            jax.ShapeDtypeStruct((1, 1), jnp.float32),
        ],
    )(seq2, bg, eg_pad, wmb, net_W, netb2, out_W, outb2, tgt_t)

    logits = jnp.transpose(logits_t, (0, 2, 1, 3))
    return logits, loss[0, 0]
